# baseline (speedup 1.0000x reference)
import functools

import jax
import jax.numpy as jnp
from jax import lax
from jax.experimental import pallas as pl
from jax.experimental.pallas import tpu as pltpu
from jax.experimental.pallas import tpu_sc as plsc

Q = 1024
N = 100000
D = 16
K_TOP = 64
CHUNK = 128
BK = 2048
N_PAD = 102400
NCHUNK = N_PAD // CHUNK
CSEL = 64
NSEL = N_PAD // CSEL
NCAND = K_TOP * CSEL
QB = 128
EPS = 1e-12
NEG = -2.0
NEG2 = -3.0

_INTERPRET = False


def _sims_body(z_ref, t_ref, sims_ref, cmax_ref):
    j = pl.program_id(0)
    z = z_ref[...]
    qn = z / jnp.maximum(jnp.sqrt(jnp.sum(z * z, axis=1, keepdims=True)), EPS)
    t = t_ref[...]
    tn = t / jnp.maximum(jnp.sqrt(jnp.sum(t * t, axis=1, keepdims=True)), EPS)
    sims = lax.dot_general(qn, tn, (((1,), (1,)), ((), ())),
                           preferred_element_type=jnp.float32)
    col = jax.lax.broadcasted_iota(jnp.int32, (Q, BK), 1) + j * BK
    sims = jnp.where(col < N, sims, NEG)
    sims_ref[...] = sims.reshape(Q, BK // CHUNK, CHUNK)
    cmax_ref[...] = jnp.max(sims.reshape(Q, BK // CSEL, CSEL), axis=-1).T


def _chunksel_body(cmax_ref, ids_ref):
    x = cmax_ref[...]
    row = jax.lax.broadcasted_iota(jnp.int32, (NSEL, Q), 0)
    out_row = jax.lax.broadcasted_iota(jnp.int32, (K_TOP, Q), 0)

    def step(i, carry):
        x, ids = carry
        m = jnp.max(x, axis=0, keepdims=True)
        pos = jnp.min(jnp.where(x == m, row, jnp.int32(1 << 30)),
                      axis=0, keepdims=True)
        ids = jnp.where(out_row == i, pos, ids)
        x = jnp.where(row == pos, NEG2, x)
        return x, ids

    _, ids = lax.fori_loop(0, K_TOP, step,
                           (x, jnp.zeros((K_TOP, Q), jnp.int32)))
    ids_ref[...] = ids


def _final_body(vals_ref, par_ref, gidx_ref, sim_out_ref, idx_out_ref):
    v3 = vals_ref[...]
    par = par_ref[...]
    v = jnp.where(par > 0, v3[:, :, CSEL:], v3[:, :, :CSEL])
    g = gidx_ref[...]
    out_lane = jax.lax.broadcasted_iota(jnp.int32, (QB, K_TOP), 1)
    big = jnp.int32(1 << 30)

    def step(i, carry):
        v, sims, idxs = carry
        m = jnp.max(jnp.max(v, axis=2), axis=1, keepdims=True)
        hit = v == m[:, :, None]
        sel = jnp.min(jnp.min(jnp.where(hit, g, big), axis=2),
                      axis=1, keepdims=True)
        v = jnp.where(hit & (g == sel[:, :, None]), NEG2, v)
        sims = jnp.where(out_lane == i, m, sims)
        idxs = jnp.where(out_lane == i, sel, idxs)
        return v, sims, idxs

    _, sims, idxs = lax.fori_loop(
        0, K_TOP, step,
        (v, jnp.zeros((QB, K_TOP), jnp.float32), jnp.zeros((QB, K_TOP), jnp.int32)))
    sim_out_ref[...] = sims
    idx_out_ref[...] = idxs


NW = 32
B_TOTAL = Q * K_TOP
BPW = B_TOTAL // NW
GCHUNK = 128

def _gather_sc_body(table_hbm, gcid_hbm, out_hbm, idx_v, rows_v, sem):
    wid = lax.axis_index("s") * 2 + lax.axis_index("c")
    base = wid * BPW
    pltpu.sync_copy(gcid_hbm.at[pl.ds(base, BPW)], idx_v)

    def body(i, carry):
        pltpu.async_copy(
            table_hbm.at[idx_v.at[pl.ds(i * GCHUNK, GCHUNK)]], rows_v, sem
        ).wait()
        pltpu.sync_copy(rows_v, out_hbm.at[pl.ds(base + i * GCHUNK, GCHUNK)])
        return carry

    lax.fori_loop(0, BPW // GCHUNK, body, 0)


def _gather_sc(table, gcid):
    mesh = plsc.VectorSubcoreMesh(core_axis_name="c", subcore_axis_name="s",
                                  num_cores=2, num_subcores=16)
    return pl.kernel(
        _gather_sc_body,
        out_type=jax.ShapeDtypeStruct((B_TOTAL, CHUNK), jnp.float32),
        mesh=mesh,
        scratch_types=[
            pltpu.VMEM((BPW,), jnp.int32),
            pltpu.VMEM((GCHUNK, CHUNK), jnp.float32),
            pltpu.SemaphoreType.DMA,
        ],
    )(table, gcid)


def kernel(z_cell, type_embeddings):
    t_pad = jnp.pad(type_embeddings, ((0, N_PAD - N), (0, 0)))

    sims, cmax_t = pl.pallas_call(
        _sims_body,
        grid=(N_PAD // BK,),
        in_specs=[
            pl.BlockSpec((Q, D), lambda j: (0, 0)),
            pl.BlockSpec((BK, D), lambda j: (j, 0)),
        ],
        out_specs=[
            pl.BlockSpec((Q, BK // CHUNK, CHUNK), lambda j: (0, j, 0)),
            pl.BlockSpec((BK // CSEL, Q), lambda j: (j, 0)),
        ],
        out_shape=[
            jax.ShapeDtypeStruct((Q, NCHUNK, CHUNK), jnp.float32),
            jax.ShapeDtypeStruct((NSEL, Q), jnp.float32),
        ],
        interpret=_INTERPRET,
    )(z_cell, t_pad)

    ids_t = pl.pallas_call(
        _chunksel_body,
        out_shape=jax.ShapeDtypeStruct((K_TOP, Q), jnp.int32),
        interpret=_INTERPRET,
    )(cmax_t)
    chunk_ids = ids_t.T

    table = sims.reshape(Q * NCHUNK, CHUNK)
    gcid = ((chunk_ids >> 1)
            + jnp.arange(Q, dtype=jnp.int32)[:, None] * NCHUNK
            ).reshape(Q * K_TOP)
    parity = (chunk_ids & 1)[:, :, None]
    gathered = _gather_sc(table, gcid)

    vals3 = gathered.reshape(Q, K_TOP, CHUNK)
    gidx = chunk_ids[:, :, None] * CSEL + jnp.arange(CSEL, dtype=jnp.int32)

    sims_out, idx_out = pl.pallas_call(
        _final_body,
        grid=(Q // QB,),
        in_specs=[
            pl.BlockSpec((QB, K_TOP, CHUNK), lambda i: (i, 0, 0)),
            pl.BlockSpec((QB, K_TOP, 1), lambda i: (i, 0, 0)),
            pl.BlockSpec((QB, K_TOP, CSEL), lambda i: (i, 0, 0)),
        ],
        out_specs=[
            pl.BlockSpec((QB, K_TOP), lambda i: (i, 0)),
            pl.BlockSpec((QB, K_TOP), lambda i: (i, 0)),
        ],
        out_shape=[
            jax.ShapeDtypeStruct((Q, K_TOP), jnp.float32),
            jax.ShapeDtypeStruct((Q, K_TOP), jnp.int32),
        ],
        interpret=_INTERPRET,
    )(vals3, parity, gidx)

    return sims_out, idx_out


# speedup vs baseline: 2.7222x; 2.7222x over previous
import functools

import jax
import jax.numpy as jnp
from jax import lax
from jax.experimental import pallas as pl
from jax.experimental.pallas import tpu as pltpu
from jax.experimental.pallas import tpu_sc as plsc

Q = 1024
N = 100000
D = 16
K_TOP = 64
CHUNK = 128
BK = 2048
N_PAD = 102400
NCHUNK = N_PAD // CHUNK
NCAND = K_TOP * CHUNK
QB = 256
EPS = 1e-12
NEG = -2.0
NEG2 = -3.0

_INTERPRET = False


def _sims_body(z_ref, t_ref, sims_ref, cmax_ref):
    j = pl.program_id(0)
    z = z_ref[...]
    qn = z / jnp.maximum(jnp.sqrt(jnp.sum(z * z, axis=1, keepdims=True)), EPS)
    t = t_ref[...]
    tn = t / jnp.maximum(jnp.sqrt(jnp.sum(t * t, axis=1, keepdims=True)), EPS)
    sims = lax.dot_general(qn, tn, (((1,), (1,)), ((), ())),
                           preferred_element_type=jnp.float32)
    col = jax.lax.broadcasted_iota(jnp.int32, (Q, BK), 1) + j * BK
    sims = jnp.where(col < N, sims, NEG)
    sims3 = sims.reshape(Q, BK // CHUNK, CHUNK)
    sims_ref[...] = sims3
    cmax_ref[...] = jnp.max(sims3, axis=-1).T


def _chunksel_body(cmax_ref, ids_ref):
    x = cmax_ref[...]
    row = jax.lax.broadcasted_iota(jnp.int32, (NCHUNK, Q), 0)
    out_row = jax.lax.broadcasted_iota(jnp.int32, (K_TOP, Q), 0)

    def step(i, carry):
        x, ids = carry
        m = jnp.max(x, axis=0, keepdims=True)
        pos = jnp.min(jnp.where(x == m, row, jnp.int32(1 << 30)),
                      axis=0, keepdims=True)
        ids = jnp.where(out_row == i, pos, ids)
        x = jnp.where(row == pos, NEG2, x)
        return x, ids

    _, ids = lax.fori_loop(0, K_TOP, step,
                           (x, jnp.zeros((K_TOP, Q), jnp.int32)))
    ids_ref[...] = ids


def _final_body(vals_ref, cids_ref, sim_out_ref, idx_out_ref):
    v = vals_ref[...]
    lane = jax.lax.broadcasted_iota(jnp.int32, (QB, NCAND), 1)
    out_lane = jax.lax.broadcasted_iota(jnp.int32, (QB, K_TOP), 1)
    big = jnp.int32(1 << 30)

    def step(i, carry):
        v, sims, poss = carry
        m = jnp.max(v, axis=1, keepdims=True)
        pos = jnp.min(jnp.where(v == m, lane, big), axis=1, keepdims=True)
        v = jnp.where(lane == pos, NEG2, v)
        sims = jnp.where(out_lane == i, m, sims)
        poss = jnp.where(out_lane == i, pos, poss)
        return v, sims, poss

    _, sims, poss = lax.fori_loop(
        0, K_TOP, step,
        (v, jnp.zeros((QB, K_TOP), jnp.float32),
         jnp.zeros((QB, K_TOP), jnp.int32)))

    cids = cids_ref[...]
    ph = poss >> 7
    sel_cid = jnp.zeros((QB, K_TOP), jnp.int32)
    for c in range(K_TOP):
        sel_cid = jnp.where(ph == c, cids[:, c:c + 1], sel_cid)
    sim_out_ref[...] = sims
    idx_out_ref[...] = sel_cid * CHUNK + (poss & (CHUNK - 1))


NW = 32
B_TOTAL = Q * K_TOP
BPW = B_TOTAL // NW
GCHUNK = 128


def _gather_sc_body(table_hbm, gcid_hbm, out_hbm, idx_v, rows_v, sem):
    wid = lax.axis_index("s") * 2 + lax.axis_index("c")
    base = wid * BPW
    pltpu.sync_copy(gcid_hbm.at[pl.ds(base, BPW)], idx_v)

    def body(i, carry):
        pltpu.async_copy(
            table_hbm.at[idx_v.at[pl.ds(i * GCHUNK, GCHUNK)]], rows_v, sem
        ).wait()
        pltpu.sync_copy(rows_v, out_hbm.at[pl.ds(base + i * GCHUNK, GCHUNK)])
        return carry

    lax.fori_loop(0, BPW // GCHUNK, body, 0)


def _gather_sc(table, gcid):
    mesh = plsc.VectorSubcoreMesh(core_axis_name="c", subcore_axis_name="s",
                                  num_cores=2, num_subcores=16)
    return pl.kernel(
        _gather_sc_body,
        out_type=jax.ShapeDtypeStruct((B_TOTAL, CHUNK), jnp.float32),
        mesh=mesh,
        scratch_types=[
            pltpu.VMEM((BPW,), jnp.int32),
            pltpu.VMEM((GCHUNK, CHUNK), jnp.float32),
            pltpu.SemaphoreType.DMA,
        ],
    )(table, gcid)


def kernel(z_cell, type_embeddings):
    t_pad = jnp.pad(type_embeddings, ((0, N_PAD - N), (0, 0)))

    sims, cmax_t = pl.pallas_call(
        _sims_body,
        grid=(N_PAD // BK,),
        in_specs=[
            pl.BlockSpec((Q, D), lambda j: (0, 0)),
            pl.BlockSpec((BK, D), lambda j: (j, 0)),
        ],
        out_specs=[
            pl.BlockSpec((Q, BK // CHUNK, CHUNK), lambda j: (0, j, 0)),
            pl.BlockSpec((BK // CHUNK, Q), lambda j: (j, 0)),
        ],
        out_shape=[
            jax.ShapeDtypeStruct((Q, NCHUNK, CHUNK), jnp.float32),
            jax.ShapeDtypeStruct((NCHUNK, Q), jnp.float32),
        ],
        interpret=_INTERPRET,
    )(z_cell, t_pad)

    ids_t = pl.pallas_call(
        _chunksel_body,
        out_shape=jax.ShapeDtypeStruct((K_TOP, Q), jnp.int32),
        interpret=_INTERPRET,
    )(cmax_t)
    chunk_ids = ids_t.T

    table = sims.reshape(Q * NCHUNK, CHUNK)
    gcid = (chunk_ids + jnp.arange(Q, dtype=jnp.int32)[:, None] * NCHUNK
            ).reshape(Q * K_TOP)
    gathered = _gather_sc(table, gcid)

    vals = gathered.reshape(Q, NCAND)

    sims_out, idx_out = pl.pallas_call(
        _final_body,
        grid=(Q // QB,),
        in_specs=[
            pl.BlockSpec((QB, NCAND), lambda i: (i, 0)),
            pl.BlockSpec((QB, K_TOP), lambda i: (i, 0)),
        ],
        out_specs=[
            pl.BlockSpec((QB, K_TOP), lambda i: (i, 0)),
            pl.BlockSpec((QB, K_TOP), lambda i: (i, 0)),
        ],
        out_shape=[
            jax.ShapeDtypeStruct((Q, K_TOP), jnp.float32),
            jax.ShapeDtypeStruct((Q, K_TOP), jnp.int32),
        ],
        interpret=_INTERPRET,
    )(vals, chunk_ids)

    return sims_out, idx_out
